# baseline (speedup 1.0000x reference)
import functools

import jax
import jax.numpy as jnp
from jax import lax
from jax.experimental import pallas as pl
from jax.experimental.pallas import tpu as pltpu
from jax.experimental.pallas import tpu_sc as plsc

S = 2048
DK = 64
H = 12
LANES = 16
NW = 32
BLK = 16
NBLK = H * (S // BLK)
BPW = NBLK // NW
COLS = H * DK
CPW = COLS // NW


def _roll0(x, s):
    s = s % x.shape[0]
    if s == 0:
        return x
    return jnp.concatenate([x[-s:], x[:-s]], axis=0)


def _sort_kernel(q_ref, k_ref, pq_ref, pk_ref, v_ref):
    q = q_ref[0, 0]
    k = k_ref[0, 0]
    keys = jnp.concatenate([q, k], axis=1)
    idx = lax.broadcasted_iota(jnp.int32, (S, 2 * DK), 0)
    row = lax.broadcasted_iota(jnp.int32, (S, 1), 0)

    for stage in range(11):
        asc = ((row >> (stage + 1)) & 1) == 0
        for sub in range(stage, -1, -1):
            j = 1 << sub
            if True:
                m = S // (2 * j)
                kr = keys.reshape(m, 2, j, 2 * DK)
                ir = idx.reshape(m, 2, j, 2 * DK)
                ka, kb = kr[:, 0], kr[:, 1]
                ia, ib = ir[:, 0], ir[:, 1]
                gi = lax.broadcasted_iota(jnp.int32, (m, 1, 1), 0)
                asc_g = ((gi >> (stage - sub)) & 1) == 0
                b_less = (kb < ka) | ((kb == ka) & (ib < ia))
                swap = b_less == asc_g
                new_ka = jnp.where(swap, kb, ka)
                new_kb = jnp.where(swap, ka, kb)
                new_ia = jnp.where(swap, ib, ia)
                new_ib = jnp.where(swap, ia, ib)
                keys = jnp.concatenate(
                    [new_ka[:, None], new_kb[:, None]], axis=1
                ).reshape(S, 2 * DK)
                idx = jnp.concatenate(
                    [new_ia[:, None], new_ib[:, None]], axis=1
                ).reshape(S, 2 * DK)
            else:
                is_lo = ((row >> sub) & 1) == 0
                want_small = is_lo == asc
                pkey = jnp.where(is_lo, _roll0(keys, -j), _roll0(keys, j))
                pidx = jnp.where(is_lo, _roll0(idx, -j), _roll0(idx, j))
                p_less = (pkey < keys) | ((pkey == keys) & (pidx < idx))
                take_p = p_less == want_small
                keys = jnp.where(take_p, pkey, keys)
                idx = jnp.where(take_p, pidx, idx)

    qs = keys[:, :DK]
    ks = keys[:, DK:]
    d = qs - ks
    v = jnp.exp(-(d * d)) * (1.0 / DK)
    pq_ref[0] = idx[:, :DK].T
    pk_ref[0] = idx[:, DK:].T
    v_ref[0] = v.T


def _tc_sort(q, k, heads=H, interpret=False):
    return pl.pallas_call(
        _sort_kernel,
        grid=(heads,),
        in_specs=[
            pl.BlockSpec((1, 1, S, DK), lambda h: (0, h, 0, 0)),
            pl.BlockSpec((1, 1, S, DK), lambda h: (0, h, 0, 0)),
        ],
        out_specs=[
            pl.BlockSpec((1, DK, S), lambda h: (h, 0, 0)),
            pl.BlockSpec((1, DK, S), lambda h: (h, 0, 0)),
            pl.BlockSpec((1, DK, S), lambda h: (h, 0, 0)),
        ],
        out_shape=[
            jax.ShapeDtypeStruct((heads, DK, S), jnp.int32),
            jax.ShapeDtypeStruct((heads, DK, S), jnp.int32),
            jax.ShapeDtypeStruct((heads, DK, S), jnp.float32),
        ],
        interpret=interpret,
    )(q, k)


def _sc_invert(pqT, pkT, vT, heads=H):
    mesh = plsc.VectorSubcoreMesh(core_axis_name="c", subcore_axis_name="s")
    cpw = heads * DK // NW

    @functools.partial(
        pl.kernel,
        mesh=mesh,
        out_type=[
            jax.ShapeDtypeStruct((heads, DK, S), jnp.int32),
            jax.ShapeDtypeStruct((heads, DK, S), jnp.float32),
        ],
        scratch_types=[
            pltpu.VMEM((S,), jnp.int32),
            pltpu.VMEM((S,), jnp.int32),
            pltpu.VMEM((S,), jnp.float32),
            pltpu.VMEM((S,), jnp.int32),
            pltpu.VMEM((S,), jnp.float32),
        ],
        compiler_params=pltpu.CompilerParams(needs_layout_passes=False, use_tc_tiling_on_sc=False),
    )
    def kern(pq_hbm, pk_hbm, v_hbm, col_hbm, w_hbm, pq_v, pk_v, v_v, col_v, w_v):
        wid = lax.axis_index("s") * 2 + lax.axis_index("c")
        for t in range(cpw):
            cc = wid * cpw + t
            h = cc // DK
            dd = cc % DK
            pltpu.sync_copy(pq_hbm.at[h, dd], pq_v)
            pltpu.sync_copy(pk_hbm.at[h, dd], pk_v)
            pltpu.sync_copy(v_hbm.at[h, dd], v_v)

            def body(i, carry):
                sl = pl.ds(i * LANES, LANES)
                pq16 = pq_v[sl]
                plsc.store_scatter(col_v, [pq16], pk_v[sl])
                plsc.store_scatter(w_v, [pq16], v_v[sl])
                return carry

            lax.fori_loop(0, S // LANES, body, 0)
            pltpu.sync_copy(col_v, col_hbm.at[h, dd])
            pltpu.sync_copy(w_v, w_hbm.at[h, dd])

    return kern(pqT, pkT, vT)


def _sc_scatter(colT, wT, mask32):
    mesh = plsc.VectorSubcoreMesh(core_axis_name="c", subcore_axis_name="s")
    RB = S // BLK

    @functools.partial(
        pl.kernel,
        mesh=mesh,
        out_type=jax.ShapeDtypeStruct((H, S, S), jnp.float32),
        scratch_types=[
            pltpu.VMEM((2, BLK, S), jnp.float32),
            pltpu.VMEM((DK, LANES), jnp.int32),
            pltpu.VMEM((DK, LANES), jnp.float32),
            pltpu.VMEM((2, DK, LANES), jnp.int32),
            pltpu.VMEM((2, DK, LANES), jnp.float32),
            pltpu.VMEM((BLK, S // 4), jnp.int32),
            pltpu.SemaphoreType.DMA,
            pltpu.SemaphoreType.DMA,
        ],
        compiler_params=pltpu.CompilerParams(needs_layout_passes=False, use_tc_tiling_on_sc=False),
    )
    def kern(col_hbm, w_hbm, mask_hbm, out_hbm,
             buf, colin, win, colst, wst, mask_v, sem0, sem1):
        wid = lax.axis_index("s") * 2 + lax.axis_index("c")
        lane = lax.iota(jnp.int32, LANES)
        zeros16 = jnp.zeros((LANES,), jnp.float32)
        sems = [sem0, sem1]

        for b2 in range(2):
            for r in range(BLK):
                def zloop(i, c, b2=b2, r=r):
                    buf[b2, r, pl.ds(i * LANES, LANES)] = zeros16
                    return c
                lax.fori_loop(0, S // LANES, zloop, 0)

        for t in range(BPW + 2):
            par = t % 2
            if t >= 2:
                bbp = wid * BPW + (t - 2)
                hp = bbp // RB
                ibp = (bbp % RB) * BLK
                pltpu.make_async_copy(
                    buf.at[par], out_hbm.at[hp, pl.ds(ibp, BLK)], sems[par]
                ).wait()
                if t < BPW:
                    def ubody(dd, c, par=par):
                        plsc.addupdate_scatter(
                            buf.at[par], [lane, colst[par, dd]],
                            -wst[par, dd])
                        return c
                    lax.fori_loop(0, DK, ubody, 0)

            if t < BPW:
                bb = wid * BPW + t
                h = bb // RB
                ib = (bb % RB) * BLK
                pltpu.sync_copy(col_hbm.at[h, :, pl.ds(ib, LANES)], colin)
                pltpu.sync_copy(w_hbm.at[h, :, pl.ds(ib, LANES)], win)
                pltpu.sync_copy(mask_hbm.at[pl.ds(ib, BLK)], mask_v)

                def abody(dd, c, par=par):
                    col16 = colin[dd]
                    w16 = win[dd]
                    word = plsc.load_gather(
                        mask_v, [lane, lax.shift_right_logical(col16, 2)])
                    sh = (col16 & 3) * 8
                    bit = lax.shift_right_logical(word, sh) & 1
                    weff = jnp.where(bit != 0, 0.0, w16)
                    plsc.addupdate_scatter(buf.at[par], [lane, col16], weff)
                    colst[par, dd] = col16
                    wst[par, dd] = weff
                    return c

                lax.fori_loop(0, DK, abody, 0)
                pltpu.async_copy(
                    buf.at[par], out_hbm.at[h, pl.ds(ib, BLK)], sems[par])

    return kern(colT, wT, mask32)


def kernel(q, k, attn_mask):
    pqT, pkT, vT = _tc_sort(q, k)
    colT, wT = _sc_invert(pqT, pkT, vT)
    m8 = attn_mask.reshape(S, S).astype(jnp.uint8).reshape(S, S // 4, 4)
    mask32 = lax.bitcast_convert_type(m8, jnp.int32)
    out = _sc_scatter(colT, wT, mask32)
    return out.reshape(1, H, S, S)


# speedup vs baseline: 2.4158x; 2.4158x over previous
import functools

import jax
import jax.numpy as jnp
from jax import lax
from jax.experimental import pallas as pl
from jax.experimental.pallas import tpu as pltpu
from jax.experimental.pallas import tpu_sc as plsc

S = 2048
DK = 64
H = 12
LANES = 16
NW = 32
BLK = 16
NBLK = H * (S // BLK)
BPW = NBLK // NW
COLS = H * DK
CPW = COLS // NW


def _roll0(x, s):
    s = s % x.shape[0]
    if s == 0:
        return x
    return jnp.concatenate([x[-s:], x[:-s]], axis=0)


def _sort_kernel(q_ref, k_ref, pq_ref, pk_ref, v_ref):
    q = q_ref[0, 0]
    k = k_ref[0, 0]
    keys = jnp.concatenate([q, k], axis=1)
    idx = lax.broadcasted_iota(jnp.int32, (S, 2 * DK), 0)
    row = lax.broadcasted_iota(jnp.int32, (S, 1), 0)

    for stage in range(11):
        asc = ((row >> (stage + 1)) & 1) == 0
        for sub in range(stage, -1, -1):
            j = 1 << sub
            if j >= 8:
                m = S // (2 * j)
                kr = keys.reshape(m, 2, j, 2 * DK)
                ir = idx.reshape(m, 2, j, 2 * DK)
                ka, kb = kr[:, 0], kr[:, 1]
                ia, ib = ir[:, 0], ir[:, 1]
                gi = lax.broadcasted_iota(jnp.int32, (m, 1, 1), 0)
                asc_g = ((gi >> (stage - sub)) & 1) == 0
                b_less = (kb < ka) | ((kb == ka) & (ib < ia))
                swap = b_less == asc_g
                new_ka = jnp.where(swap, kb, ka)
                new_kb = jnp.where(swap, ka, kb)
                new_ia = jnp.where(swap, ib, ia)
                new_ib = jnp.where(swap, ia, ib)
                keys = jnp.concatenate(
                    [new_ka[:, None], new_kb[:, None]], axis=1
                ).reshape(S, 2 * DK)
                idx = jnp.concatenate(
                    [new_ia[:, None], new_ib[:, None]], axis=1
                ).reshape(S, 2 * DK)
            else:
                is_lo = ((row >> sub) & 1) == 0
                want_small = is_lo == asc
                pkey = jnp.where(is_lo, _roll0(keys, -j), _roll0(keys, j))
                pidx = jnp.where(is_lo, _roll0(idx, -j), _roll0(idx, j))
                p_less = (pkey < keys) | ((pkey == keys) & (pidx < idx))
                take_p = p_less == want_small
                keys = jnp.where(take_p, pkey, keys)
                idx = jnp.where(take_p, pidx, idx)

    qs = keys[:, :DK]
    ks = keys[:, DK:]
    d = qs - ks
    v = jnp.exp(-(d * d)) * (1.0 / DK)
    pq_ref[0] = idx[:, :DK].T
    pk_ref[0] = idx[:, DK:].T
    v_ref[0] = v.T


def _tc_sort(q, k, heads=H, interpret=False):
    return pl.pallas_call(
        _sort_kernel,
        grid=(heads,),
        in_specs=[
            pl.BlockSpec((1, 1, S, DK), lambda h: (0, h, 0, 0)),
            pl.BlockSpec((1, 1, S, DK), lambda h: (0, h, 0, 0)),
        ],
        out_specs=[
            pl.BlockSpec((1, DK, S), lambda h: (h, 0, 0)),
            pl.BlockSpec((1, DK, S), lambda h: (h, 0, 0)),
            pl.BlockSpec((1, DK, S), lambda h: (h, 0, 0)),
        ],
        out_shape=[
            jax.ShapeDtypeStruct((heads, DK, S), jnp.int32),
            jax.ShapeDtypeStruct((heads, DK, S), jnp.int32),
            jax.ShapeDtypeStruct((heads, DK, S), jnp.float32),
        ],
        interpret=interpret,
    )(q, k)


def _sc_invert(pqT, pkT, vT, heads=H):
    mesh = plsc.VectorSubcoreMesh(core_axis_name="c", subcore_axis_name="s")
    cpw = heads * DK // NW

    @functools.partial(
        pl.kernel,
        mesh=mesh,
        out_type=[
            jax.ShapeDtypeStruct((heads, DK, S), jnp.int32),
            jax.ShapeDtypeStruct((heads, DK, S), jnp.float32),
        ],
        scratch_types=[
            pltpu.VMEM((S,), jnp.int32),
            pltpu.VMEM((S,), jnp.int32),
            pltpu.VMEM((S,), jnp.float32),
            pltpu.VMEM((S,), jnp.int32),
            pltpu.VMEM((S,), jnp.float32),
        ],
        compiler_params=pltpu.CompilerParams(needs_layout_passes=False, use_tc_tiling_on_sc=False),
    )
    def kern(pq_hbm, pk_hbm, v_hbm, col_hbm, w_hbm, pq_v, pk_v, v_v, col_v, w_v):
        wid = lax.axis_index("s") * 2 + lax.axis_index("c")
        for t in range(cpw):
            cc = wid * cpw + t
            h = cc // DK
            dd = cc % DK
            pltpu.sync_copy(pq_hbm.at[h, dd], pq_v)
            pltpu.sync_copy(pk_hbm.at[h, dd], pk_v)
            pltpu.sync_copy(v_hbm.at[h, dd], v_v)

            def body(i, carry):
                sl = pl.ds(i * LANES, LANES)
                pq16 = pq_v[sl]
                plsc.store_scatter(col_v, [pq16], pk_v[sl])
                plsc.store_scatter(w_v, [pq16], v_v[sl])
                return carry

            lax.fori_loop(0, S // LANES, body, 0)
            pltpu.sync_copy(col_v, col_hbm.at[h, dd])
            pltpu.sync_copy(w_v, w_hbm.at[h, dd])

    return kern(pqT, pkT, vT)


def _sc_scatter(colT, wT, mask32):
    mesh = plsc.VectorSubcoreMesh(core_axis_name="c", subcore_axis_name="s")
    RB = S // BLK

    @functools.partial(
        pl.kernel,
        mesh=mesh,
        out_type=jax.ShapeDtypeStruct((H, S, S), jnp.float32),
        scratch_types=[
            pltpu.VMEM((2, BLK, S), jnp.float32),
            pltpu.VMEM((DK, LANES), jnp.int32),
            pltpu.VMEM((DK, LANES), jnp.float32),
            pltpu.VMEM((2, DK, LANES), jnp.int32),
            pltpu.VMEM((2, DK, LANES), jnp.float32),
            pltpu.VMEM((BLK, S // 4), jnp.int32),
            pltpu.SemaphoreType.DMA,
            pltpu.SemaphoreType.DMA,
        ],
        compiler_params=pltpu.CompilerParams(needs_layout_passes=False, use_tc_tiling_on_sc=False),
    )
    def kern(col_hbm, w_hbm, mask_hbm, out_hbm,
             buf, colin, win, colst, wst, mask_v, sem0, sem1):
        wid = lax.axis_index("s") * 2 + lax.axis_index("c")
        lane = lax.iota(jnp.int32, LANES)
        zeros16 = jnp.zeros((LANES,), jnp.float32)
        sems = [sem0, sem1]

        for b2 in range(2):
            for r in range(BLK):
                def zloop(i, c, b2=b2, r=r):
                    buf[b2, r, pl.ds(i * LANES, LANES)] = zeros16
                    return c
                lax.fori_loop(0, S // LANES, zloop, 0)

        for t in range(BPW + 2):
            par = t % 2
            if t >= 2:
                bbp = wid * BPW + (t - 2)
                hp = bbp // RB
                ibp = (bbp % RB) * BLK
                pltpu.make_async_copy(
                    buf.at[par], out_hbm.at[hp, pl.ds(ibp, BLK)], sems[par]
                ).wait()
                if t < BPW:
                    def ubody(dd, c, par=par):
                        plsc.addupdate_scatter(
                            buf.at[par], [lane, colst[par, dd]],
                            -wst[par, dd])
                        return c
                    lax.fori_loop(0, DK, ubody, 0)

            if t < BPW:
                bb = wid * BPW + t
                h = bb // RB
                ib = (bb % RB) * BLK
                pltpu.sync_copy(col_hbm.at[h, :, pl.ds(ib, LANES)], colin)
                pltpu.sync_copy(w_hbm.at[h, :, pl.ds(ib, LANES)], win)
                pltpu.sync_copy(mask_hbm.at[pl.ds(ib, BLK)], mask_v)

                def abody(dd, c, par=par):
                    col16 = colin[dd]
                    w16 = win[dd]
                    word = plsc.load_gather(
                        mask_v, [lane, lax.shift_right_logical(col16, 2)])
                    sh = (col16 & 3) * 8
                    bit = lax.shift_right_logical(word, sh) & 1
                    weff = jnp.where(bit != 0, 0.0, w16)
                    plsc.addupdate_scatter(buf.at[par], [lane, col16], weff)
                    colst[par, dd] = col16
                    wst[par, dd] = weff
                    return c

                lax.fori_loop(0, DK, abody, 0)
                pltpu.async_copy(
                    buf.at[par], out_hbm.at[h, pl.ds(ib, BLK)], sems[par])

    return kern(colT, wT, mask32)


def kernel(q, k, attn_mask):
    pqT, pkT, vT = _tc_sort(q, k)
    colT, wT = _sc_invert(pqT, pkT, vT)
    m8 = attn_mask.reshape(S, S).astype(jnp.uint8).reshape(S, S // 4, 4)
    mask32 = lax.bitcast_convert_type(m8, jnp.int32)
    out = _sc_scatter(colT, wT, mask32)
    return out.reshape(1, H, S, S)
